# baseline (speedup 1.0000x reference)
import functools

import jax
import jax.numpy as jnp
from jax import lax
from jax.experimental import pallas as pl
from jax.experimental.pallas import tpu as pltpu
from jax.experimental.pallas import tpu_sc as plsc

N = 16384
D_IN = 1024
D = 512
G = 4
M = N // G
L = 16
BL = 512
BK = 2048


def _dense_body(x_ref, w1_ref, v_ref, wvt_ref, wct_ref,
                proj_ref, s_ref, dc_ref):
    p = jnp.maximum(
        jnp.dot(x_ref[...], w1_ref[...], preferred_element_type=jnp.float32),
        0.0)
    proj_ref[...] = p.astype(jnp.bfloat16)
    t = jnp.tanh(jnp.dot(p, v_ref[...], preferred_element_type=jnp.float32))
    s_ref[...] = jnp.sum(t * wvt_ref[...], axis=1)
    dc_ref[...] = jnp.sum(p * wct_ref[...], axis=1)


def _dense_pass(tfeat, W1, V, WvT, WcT):
    return pl.pallas_call(
        _dense_body,
        grid=(N // BL,),
        in_specs=[
            pl.BlockSpec((BL, D_IN), lambda i: (i, 0)),
            pl.BlockSpec((D_IN, D), lambda i: (0, 0)),
            pl.BlockSpec((D, D), lambda i: (0, 0)),
            pl.BlockSpec((1, D), lambda i: (0, 0)),
            pl.BlockSpec((1, D), lambda i: (0, 0)),
        ],
        out_specs=[
            pl.BlockSpec((BL, D), lambda i: (i, 0)),
            pl.BlockSpec((BL,), lambda i: (i,)),
            pl.BlockSpec((BL,), lambda i: (i,)),
        ],
        out_shape=[
            jax.ShapeDtypeStruct((N, D), jnp.bfloat16),
            jax.ShapeDtypeStruct((N,), jnp.float32),
            jax.ShapeDtypeStruct((N,), jnp.float32),
        ],
    )(tfeat, W1, V, WvT, WcT)


def _lane_gather(x, perm):
    return lax.gather(
        x, perm[:, None],
        dimension_numbers=lax.GatherDimensionNumbers(
            offset_dims=(), collapsed_slice_dims=(0,), start_index_map=(0,)),
        slice_sizes=(1,),
        mode=lax.GatherScatterMode.PROMISE_IN_BOUNDS)


def _butterfly(x, op):
    lanes = lax.iota(jnp.int32, L)
    for sh in (8, 4, 2, 1):
        x = op(x, _lane_gather(x, (lanes + sh) % L))
    return x


def _bmax(x):
    return _butterfly(x, jnp.maximum)


def _bmin(x):
    return _butterfly(x, jnp.minimum)


def _bsum(x):
    return _butterfly(x, lax.add)


WPG = 8
B = M // WPG
BCH = B // L
NSUB = 16


def _sc_body(idx_hbm, s_hbm, dc_hbm,
             w_hbm, ids_hbm,
             idx_v, s_v, dc_v, e_v, zero_v, stg_v, stg_i, comb_v, comb_i,
             w0_sh, w1_sh, red_max, red_sum,
             red_dmax, red_rmax, red_dmin, red_rmin, sem):
    c = lax.axis_index("c")
    s = lax.axis_index("s")
    g = 2 * c + s // WPG
    slot = s % WPG
    grow = s // WPG
    base = grow * WPG * L

    def zb(j, carry):
        zero_v[pl.ds(j * L, L)] = jnp.zeros((L,), jnp.float32)
        return carry
    lax.fori_loop(0, (N // NSUB) // L, zb, 0, unroll=8)
    pltpu.sync_copy(zero_v, w0_sh.at[pl.ds(s * (N // NSUB), N // NSUB)])
    pltpu.sync_copy(zero_v, w1_sh.at[pl.ds(s * (N // NSUB), N // NSUB)])

    pltpu.sync_copy(idx_hbm.at[g, pl.ds(slot * B, B)], idx_v)
    cp_s = pltpu.async_copy(s_hbm.at[idx_v], s_v, sem)
    cp_dc = pltpu.async_copy(dc_hbm.at[idx_v], dc_v, sem)
    cp_s.wait()
    cp_dc.wait()

    def maxbody(j, mx):
        return jnp.maximum(mx, s_v[pl.ds(j * L, L)])
    lmx = lax.fori_loop(0, BCH, maxbody,
                        jnp.full((L,), -jnp.inf, jnp.float32), unroll=8)
    stg_v[...] = lmx
    pltpu.sync_copy(stg_v, red_max.at[pl.ds(s * L, L)])
    plsc.subcore_barrier()
    pltpu.sync_copy(red_max.at[pl.ds(base, WPG * L)], comb_v)
    mxv = comb_v[pl.ds(0, L)]
    for i in range(1, WPG):
        mxv = jnp.maximum(mxv, comb_v[pl.ds(i * L, L)])
    mx = _bmax(mxv)

    def expbody(j, acc):
        e = jnp.exp(s_v[pl.ds(j * L, L)] - mx)
        e_v[pl.ds(j * L, L)] = e
        return acc + e
    lsum = lax.fori_loop(0, BCH, expbody, jnp.zeros((L,), jnp.float32),
                         unroll=8)
    stg_v[...] = lsum
    pltpu.sync_copy(stg_v, red_sum.at[pl.ds(s * L, L)])
    plsc.subcore_barrier()
    pltpu.sync_copy(red_sum.at[pl.ds(base, WPG * L)], comb_v)
    zv = comb_v[pl.ds(0, L)]
    for i in range(1, WPG):
        zv = zv + comb_v[pl.ds(i * L, L)]
    inv_z = 1.0 / _bsum(zv)

    def selbody(j, carry):
        bmaxd, bmaxr, bmind, bminr = carry
        idxc = idx_v[pl.ds(j * L, L)]
        a = e_v[pl.ds(j * L, L)] * inv_z
        e_v[pl.ds(j * L, L)] = a
        d = a * dc_v[pl.ds(j * L, L)]
        upmax = d > bmaxd
        upmin = d <= bmind
        return (jnp.where(upmax, d, bmaxd), jnp.where(upmax, idxc, bmaxr),
                jnp.where(upmin, d, bmind), jnp.where(upmin, idxc, bminr))

    init = (jnp.full((L,), -jnp.inf, jnp.float32),
            jnp.zeros((L,), jnp.int32),
            jnp.full((L,), jnp.inf, jnp.float32),
            jnp.zeros((L,), jnp.int32))
    bmaxd, bmaxr, bmind, bminr = lax.fori_loop(0, BCH, selbody, init,
                                               unroll=8)
    stg_v[...] = bmaxd
    pltpu.sync_copy(stg_v, red_dmax.at[pl.ds(s * L, L)])
    stg_i[...] = bmaxr
    pltpu.sync_copy(stg_i, red_rmax.at[pl.ds(s * L, L)])
    stg_v[...] = bmind
    pltpu.sync_copy(stg_v, red_dmin.at[pl.ds(s * L, L)])
    stg_i[...] = bminr
    pltpu.sync_copy(stg_i, red_rmin.at[pl.ds(s * L, L)])

    @pl.when(grow == 0)
    def _():
        pltpu.sync_copy(e_v, w0_sh.at[idx_v], add=True)

    @pl.when(grow == 1)
    def _():
        pltpu.sync_copy(e_v, w1_sh.at[idx_v], add=True)

    plsc.subcore_barrier()

    @pl.when(slot == 0)
    def _():
        pltpu.sync_copy(red_dmax.at[pl.ds(base, WPG * L)], comb_v)
        pltpu.sync_copy(red_rmax.at[pl.ds(base, WPG * L)], comb_i)
        bd = comb_v[pl.ds(0, L)]
        br = comb_i[pl.ds(0, L)]
        for i in range(1, WPG):
            di = comb_v[pl.ds(i * L, L)]
            ri = comb_i[pl.ds(i * L, L)]
            up = di > bd
            bd = jnp.where(up, di, bd)
            br = jnp.where(up, ri, br)
        dM = _bmax(bd)
        rmax = _bmax(jnp.where(bd == dM, br, 0))

        pltpu.sync_copy(red_dmin.at[pl.ds(base, WPG * L)], comb_v)
        pltpu.sync_copy(red_rmin.at[pl.ds(base, WPG * L)], comb_i)
        bd2 = comb_v[pl.ds(0, L)]
        br2 = comb_i[pl.ds(0, L)]
        for i in range(1, WPG):
            di = comb_v[pl.ds(i * L, L)]
            ri = comb_i[pl.ds(i * L, L)]
            up = di <= bd2
            bd2 = jnp.where(up, di, bd2)
            br2 = jnp.where(up, ri, br2)
        dm = _bmin(bd2)
        rmin = _bmax(jnp.where(bd2 == dm, br2, 0))

        lanes = lax.iota(jnp.int32, L)
        iv = jnp.where(lanes == 0, rmax, rmin)
        stg_i[...] = iv
        pltpu.sync_copy(stg_i, ids_hbm.at[g])

    CW = N // WPG

    @pl.when(grow == 0)
    def _():
        pltpu.sync_copy(w0_sh.at[pl.ds(slot * CW, CW)],
                        w_hbm.at[g, pl.ds(slot * CW, CW)])

    @pl.when(grow == 1)
    def _():
        pltpu.sync_copy(w1_sh.at[pl.ds(slot * CW, CW)],
                        w_hbm.at[g, pl.ds(slot * CW, CW)])


@functools.lru_cache(maxsize=1)
def _sc_call():
    return pl.kernel(
        _sc_body,
        out_type=[
            jax.ShapeDtypeStruct((G, N), jnp.float32),
            jax.ShapeDtypeStruct((G, L), jnp.int32),
        ],
        mesh=plsc.VectorSubcoreMesh(core_axis_name="c", subcore_axis_name="s"),
        compiler_params=pltpu.CompilerParams(needs_layout_passes=False),
        scratch_types=[
            pltpu.VMEM((B,), jnp.int32),
            pltpu.VMEM((B,), jnp.float32),
            pltpu.VMEM((B,), jnp.float32),
            pltpu.VMEM((B,), jnp.float32),
            pltpu.VMEM((N // NSUB,), jnp.float32),
            pltpu.VMEM((L,), jnp.float32),
            pltpu.VMEM((L,), jnp.int32),
            pltpu.VMEM((WPG * L,), jnp.float32),
            pltpu.VMEM((WPG * L,), jnp.int32),
            pltpu.VMEM_SHARED((N,), jnp.float32),
            pltpu.VMEM_SHARED((N,), jnp.float32),
            pltpu.VMEM_SHARED((NSUB * L,), jnp.float32),
            pltpu.VMEM_SHARED((NSUB * L,), jnp.float32),
            pltpu.VMEM_SHARED((NSUB * L,), jnp.float32),
            pltpu.VMEM_SHARED((NSUB * L,), jnp.int32),
            pltpu.VMEM_SHARED((NSUB * L,), jnp.float32),
            pltpu.VMEM_SHARED((NSUB * L,), jnp.int32),
            pltpu.SemaphoreType.DMA,
        ],
    )


def _contract_body(w_ref, proj_ref, wc_ref, bc_ref, out_ref, acc_ref):
    k = pl.program_id(0)

    @pl.when(k == 0)
    def _():
        acc_ref[...] = jnp.zeros_like(acc_ref)

    acc_ref[...] += jnp.dot(w_ref[...].astype(jnp.bfloat16), proj_ref[...],
                            preferred_element_type=jnp.float32)

    @pl.when(k == pl.num_programs(0) - 1)
    def _():
        out_ref[...] = (jnp.dot(acc_ref[...], wc_ref[...],
                                preferred_element_type=jnp.float32)
                        + bc_ref[...])


def _contract_pass(w, proj, Wc, bc2):
    return pl.pallas_call(
        _contract_body,
        grid=(N // BK,),
        in_specs=[
            pl.BlockSpec((G, BK), lambda k: (0, k)),
            pl.BlockSpec((BK, D), lambda k: (k, 0)),
            pl.BlockSpec((D, 2), lambda k: (0, 0)),
            pl.BlockSpec((1, 2), lambda k: (0, 0)),
        ],
        out_specs=pl.BlockSpec((G, 2), lambda k: (0, 0)),
        out_shape=jax.ShapeDtypeStruct((G, 2), jnp.float32),
        scratch_shapes=[pltpu.VMEM((G, D), jnp.float32)],
    )(w, proj, Wc, bc2)


def _pseudo_body(ids_ref, tf_ref, w1_ref, o_ref, rows_v, sem):
    for i in range(2 * G):
        pltpu.make_async_copy(
            tf_ref.at[pl.ds(ids_ref[i], 1)],
            rows_v.at[pl.ds(i, 1)], sem).start()
    for i in range(2 * G):
        pltpu.make_async_copy(
            tf_ref.at[pl.ds(ids_ref[i], 1)],
            rows_v.at[pl.ds(i, 1)], sem).wait()
    o_ref[...] = jnp.maximum(
        jnp.dot(rows_v[...], w1_ref[...], preferred_element_type=jnp.float32),
        0.0)


def _pseudo_pass(ids, tfeat, W1):
    return pl.pallas_call(
        _pseudo_body,
        in_specs=[
            pl.BlockSpec(memory_space=pltpu.SMEM),
            pl.BlockSpec(memory_space=pltpu.HBM),
            pl.BlockSpec((D_IN, D), lambda: (0, 0)),
        ],
        out_specs=pl.BlockSpec((2 * G, D), lambda: (0, 0)),
        out_shape=jax.ShapeDtypeStruct((2 * G, D), jnp.float32),
        scratch_shapes=[pltpu.VMEM((2 * G, D_IN), jnp.float32),
                        pltpu.SemaphoreType.DMA],
    )(ids, tfeat, W1)


def kernel(tfeat_tensor, idx0, idx1, idx2, idx3, W1, V, Wv, Wc, bc):
    wdiff = (Wc[:, 1] - Wc[:, 0]).reshape(1, D)
    proj, s, dc = _dense_pass(tfeat_tensor, W1, V, Wv.T, wdiff)
    idx_all = jnp.stack([idx0, idx1, idx2, idx3])
    w, ids = _sc_call()(idx_all, s, dc)
    preds = _contract_pass(w, proj, Wc, bc.reshape(1, 2))
    pseudo = _pseudo_pass(ids[:, :2].reshape(2 * G), tfeat_tensor, W1)
    return (preds, pseudo.reshape(G, 2, D))


# speedup vs baseline: 1.0269x; 1.0269x over previous
import functools

import jax
import jax.numpy as jnp
from jax import lax
from jax.experimental import pallas as pl
from jax.experimental.pallas import tpu as pltpu
from jax.experimental.pallas import tpu_sc as plsc

N = 16384
D_IN = 1024
D = 512
G = 4
M = N // G
L = 16
BL = 512
BK = 2048


def _dense_body(x_ref, w1_ref, v_ref, wvt_ref, wct_ref,
                proj_ref, s_ref, dc_ref):
    p = jnp.maximum(
        jnp.dot(x_ref[...], w1_ref[...], preferred_element_type=jnp.float32),
        0.0)
    proj_ref[...] = p.astype(jnp.bfloat16)
    t = jnp.tanh(jnp.dot(p, v_ref[...], preferred_element_type=jnp.float32))
    s_ref[...] = jnp.sum(t * wvt_ref[...], axis=1)
    dc_ref[...] = jnp.sum(p * wct_ref[...], axis=1)


def _dense_pass(tfeat, W1, V, WvT, WcT):
    return pl.pallas_call(
        _dense_body,
        grid=(N // BL,),
        in_specs=[
            pl.BlockSpec((BL, D_IN), lambda i: (i, 0)),
            pl.BlockSpec((D_IN, D), lambda i: (0, 0)),
            pl.BlockSpec((D, D), lambda i: (0, 0)),
            pl.BlockSpec((1, D), lambda i: (0, 0)),
            pl.BlockSpec((1, D), lambda i: (0, 0)),
        ],
        out_specs=[
            pl.BlockSpec((BL, D), lambda i: (i, 0)),
            pl.BlockSpec((BL,), lambda i: (i,)),
            pl.BlockSpec((BL,), lambda i: (i,)),
        ],
        out_shape=[
            jax.ShapeDtypeStruct((N, D), jnp.bfloat16),
            jax.ShapeDtypeStruct((N,), jnp.float32),
            jax.ShapeDtypeStruct((N,), jnp.float32),
        ],
    )(tfeat, W1, V, WvT, WcT)


def _lane_gather(x, perm):
    return lax.gather(
        x, perm[:, None],
        dimension_numbers=lax.GatherDimensionNumbers(
            offset_dims=(), collapsed_slice_dims=(0,), start_index_map=(0,)),
        slice_sizes=(1,),
        mode=lax.GatherScatterMode.PROMISE_IN_BOUNDS)


def _butterfly(x, op):
    lanes = lax.iota(jnp.int32, L)
    for sh in (8, 4, 2, 1):
        x = op(x, _lane_gather(x, (lanes + sh) % L))
    return x


def _bmax(x):
    return _butterfly(x, jnp.maximum)


def _bmin(x):
    return _butterfly(x, jnp.minimum)


def _bsum(x):
    return _butterfly(x, lax.add)


WPG = 8
B = M // WPG
BCH = B // L
NSUB = 16


def _sc_body(idx0_hbm, idx1_hbm, idx2_hbm, idx3_hbm, s_hbm, dc_hbm,
             w_hbm, ids_hbm,
             idx_v, s_v, dc_v, e_v, zero_v, stg_v, stg_i, comb_v, comb_i,
             w0_sh, w1_sh, red_max, red_sum,
             red_dmax, red_rmax, red_dmin, red_rmin, sem):
    c = lax.axis_index("c")
    s = lax.axis_index("s")
    g = 2 * c + s // WPG
    slot = s % WPG
    grow = s // WPG
    base = grow * WPG * L

    def zb(j, carry):
        zero_v[pl.ds(j * L, L)] = jnp.zeros((L,), jnp.float32)
        return carry
    lax.fori_loop(0, (N // NSUB) // L, zb, 0, unroll=8)
    pltpu.sync_copy(zero_v, w0_sh.at[pl.ds(s * (N // NSUB), N // NSUB)])
    pltpu.sync_copy(zero_v, w1_sh.at[pl.ds(s * (N // NSUB), N // NSUB)])

    for gi, ref in enumerate((idx0_hbm, idx1_hbm, idx2_hbm, idx3_hbm)):
        @pl.when(g == gi)
        def _(ref=ref):
            pltpu.sync_copy(ref.at[pl.ds(slot * B, B)], idx_v)
    cp_s = pltpu.async_copy(s_hbm.at[idx_v], s_v, sem)
    cp_dc = pltpu.async_copy(dc_hbm.at[idx_v], dc_v, sem)
    cp_s.wait()
    cp_dc.wait()

    def maxbody(j, mx):
        return jnp.maximum(mx, s_v[pl.ds(j * L, L)])
    lmx = lax.fori_loop(0, BCH, maxbody,
                        jnp.full((L,), -jnp.inf, jnp.float32), unroll=8)
    stg_v[...] = lmx
    pltpu.sync_copy(stg_v, red_max.at[pl.ds(s * L, L)])
    plsc.subcore_barrier()
    pltpu.sync_copy(red_max.at[pl.ds(base, WPG * L)], comb_v)
    mxv = comb_v[pl.ds(0, L)]
    for i in range(1, WPG):
        mxv = jnp.maximum(mxv, comb_v[pl.ds(i * L, L)])
    mx = _bmax(mxv)

    def expbody(j, acc):
        e = jnp.exp(s_v[pl.ds(j * L, L)] - mx)
        e_v[pl.ds(j * L, L)] = e
        return acc + e
    lsum = lax.fori_loop(0, BCH, expbody, jnp.zeros((L,), jnp.float32),
                         unroll=8)
    stg_v[...] = lsum
    pltpu.sync_copy(stg_v, red_sum.at[pl.ds(s * L, L)])
    plsc.subcore_barrier()
    pltpu.sync_copy(red_sum.at[pl.ds(base, WPG * L)], comb_v)
    zv = comb_v[pl.ds(0, L)]
    for i in range(1, WPG):
        zv = zv + comb_v[pl.ds(i * L, L)]
    inv_z = 1.0 / _bsum(zv)

    def selbody(j, carry):
        bmaxd, bmaxr, bmind, bminr = carry
        idxc = idx_v[pl.ds(j * L, L)]
        a = e_v[pl.ds(j * L, L)] * inv_z
        e_v[pl.ds(j * L, L)] = a
        d = a * dc_v[pl.ds(j * L, L)]
        upmax = d > bmaxd
        upmin = d <= bmind
        return (jnp.where(upmax, d, bmaxd), jnp.where(upmax, idxc, bmaxr),
                jnp.where(upmin, d, bmind), jnp.where(upmin, idxc, bminr))

    init = (jnp.full((L,), -jnp.inf, jnp.float32),
            jnp.zeros((L,), jnp.int32),
            jnp.full((L,), jnp.inf, jnp.float32),
            jnp.zeros((L,), jnp.int32))
    bmaxd, bmaxr, bmind, bminr = lax.fori_loop(0, BCH, selbody, init,
                                               unroll=8)
    stg_v[...] = bmaxd
    pltpu.sync_copy(stg_v, red_dmax.at[pl.ds(s * L, L)])
    stg_i[...] = bmaxr
    pltpu.sync_copy(stg_i, red_rmax.at[pl.ds(s * L, L)])
    stg_v[...] = bmind
    pltpu.sync_copy(stg_v, red_dmin.at[pl.ds(s * L, L)])
    stg_i[...] = bminr
    pltpu.sync_copy(stg_i, red_rmin.at[pl.ds(s * L, L)])

    @pl.when(grow == 0)
    def _():
        pltpu.sync_copy(e_v, w0_sh.at[idx_v], add=True)

    @pl.when(grow == 1)
    def _():
        pltpu.sync_copy(e_v, w1_sh.at[idx_v], add=True)

    plsc.subcore_barrier()

    @pl.when(slot == 0)
    def _():
        pltpu.sync_copy(red_dmax.at[pl.ds(base, WPG * L)], comb_v)
        pltpu.sync_copy(red_rmax.at[pl.ds(base, WPG * L)], comb_i)
        bd = comb_v[pl.ds(0, L)]
        br = comb_i[pl.ds(0, L)]
        for i in range(1, WPG):
            di = comb_v[pl.ds(i * L, L)]
            ri = comb_i[pl.ds(i * L, L)]
            up = di > bd
            bd = jnp.where(up, di, bd)
            br = jnp.where(up, ri, br)
        dM = _bmax(bd)
        rmax = _bmax(jnp.where(bd == dM, br, 0))

        pltpu.sync_copy(red_dmin.at[pl.ds(base, WPG * L)], comb_v)
        pltpu.sync_copy(red_rmin.at[pl.ds(base, WPG * L)], comb_i)
        bd2 = comb_v[pl.ds(0, L)]
        br2 = comb_i[pl.ds(0, L)]
        for i in range(1, WPG):
            di = comb_v[pl.ds(i * L, L)]
            ri = comb_i[pl.ds(i * L, L)]
            up = di <= bd2
            bd2 = jnp.where(up, di, bd2)
            br2 = jnp.where(up, ri, br2)
        dm = _bmin(bd2)
        rmin = _bmax(jnp.where(bd2 == dm, br2, 0))

        lanes = lax.iota(jnp.int32, L)
        iv = jnp.where(lanes == 0, rmax, rmin)
        stg_i[...] = iv
        pltpu.sync_copy(stg_i, ids_hbm.at[g])

    CW = N // WPG

    @pl.when(grow == 0)
    def _():
        pltpu.sync_copy(w0_sh.at[pl.ds(slot * CW, CW)],
                        w_hbm.at[g, pl.ds(slot * CW, CW)])

    @pl.when(grow == 1)
    def _():
        pltpu.sync_copy(w1_sh.at[pl.ds(slot * CW, CW)],
                        w_hbm.at[g, pl.ds(slot * CW, CW)])


@functools.lru_cache(maxsize=1)
def _sc_call():
    return pl.kernel(
        _sc_body,
        out_type=[
            jax.ShapeDtypeStruct((G, N), jnp.float32),
            jax.ShapeDtypeStruct((G, L), jnp.int32),
        ],
        mesh=plsc.VectorSubcoreMesh(core_axis_name="c", subcore_axis_name="s"),
        compiler_params=pltpu.CompilerParams(needs_layout_passes=False),
        scratch_types=[
            pltpu.VMEM((B,), jnp.int32),
            pltpu.VMEM((B,), jnp.float32),
            pltpu.VMEM((B,), jnp.float32),
            pltpu.VMEM((B,), jnp.float32),
            pltpu.VMEM((N // NSUB,), jnp.float32),
            pltpu.VMEM((L,), jnp.float32),
            pltpu.VMEM((L,), jnp.int32),
            pltpu.VMEM((WPG * L,), jnp.float32),
            pltpu.VMEM((WPG * L,), jnp.int32),
            pltpu.VMEM_SHARED((N,), jnp.float32),
            pltpu.VMEM_SHARED((N,), jnp.float32),
            pltpu.VMEM_SHARED((NSUB * L,), jnp.float32),
            pltpu.VMEM_SHARED((NSUB * L,), jnp.float32),
            pltpu.VMEM_SHARED((NSUB * L,), jnp.float32),
            pltpu.VMEM_SHARED((NSUB * L,), jnp.int32),
            pltpu.VMEM_SHARED((NSUB * L,), jnp.float32),
            pltpu.VMEM_SHARED((NSUB * L,), jnp.int32),
            pltpu.SemaphoreType.DMA,
        ],
    )


def _contract_body(w_ref, proj_ref, wc_ref, bc_ref, out_ref, acc_ref):
    k = pl.program_id(0)

    @pl.when(k == 0)
    def _():
        acc_ref[...] = jnp.zeros_like(acc_ref)

    acc_ref[...] += jnp.dot(w_ref[...], proj_ref[...].astype(jnp.float32),
                            preferred_element_type=jnp.float32)

    @pl.when(k == pl.num_programs(0) - 1)
    def _():
        out_ref[...] = (jnp.dot(acc_ref[...], wc_ref[...],
                                preferred_element_type=jnp.float32)
                        + bc_ref[...])


def _contract_pass(w, proj, Wc, bc2):
    return pl.pallas_call(
        _contract_body,
        grid=(N // BK,),
        in_specs=[
            pl.BlockSpec((G, BK), lambda k: (0, k)),
            pl.BlockSpec((BK, D), lambda k: (k, 0)),
            pl.BlockSpec((D, 2), lambda k: (0, 0)),
            pl.BlockSpec((1, 2), lambda k: (0, 0)),
        ],
        out_specs=pl.BlockSpec((G, 2), lambda k: (0, 0)),
        out_shape=jax.ShapeDtypeStruct((G, 2), jnp.float32),
        scratch_shapes=[pltpu.VMEM((G, D), jnp.float32)],
    )(w, proj, Wc, bc2)


def _pseudo_body(ids_ref, tf_ref, w1_ref, o_ref, rows_v, sem):
    for i in range(2 * G):
        pltpu.make_async_copy(
            tf_ref.at[pl.ds(ids_ref[i], 1)],
            rows_v.at[pl.ds(i, 1)], sem).start()
    for i in range(2 * G):
        pltpu.make_async_copy(
            tf_ref.at[pl.ds(ids_ref[i], 1)],
            rows_v.at[pl.ds(i, 1)], sem).wait()
    res = jnp.maximum(
        jnp.dot(rows_v[...], w1_ref[...], preferred_element_type=jnp.float32),
        0.0)
    o_ref[...] = res.reshape(G, 2, D)


def _pseudo_pass(ids, tfeat, W1):
    return pl.pallas_call(
        _pseudo_body,
        in_specs=[
            pl.BlockSpec(memory_space=pltpu.SMEM),
            pl.BlockSpec(memory_space=pltpu.HBM),
            pl.BlockSpec((D_IN, D), lambda: (0, 0)),
        ],
        out_specs=pl.BlockSpec((G, 2, D), lambda: (0, 0, 0)),
        out_shape=jax.ShapeDtypeStruct((G, 2, D), jnp.float32),
        scratch_shapes=[pltpu.VMEM((2 * G, D_IN), jnp.float32),
                        pltpu.SemaphoreType.DMA],
    )(ids, tfeat, W1)


def kernel(tfeat_tensor, idx0, idx1, idx2, idx3, W1, V, Wv, Wc, bc):
    wdiff = (Wc[:, 1] - Wc[:, 0]).reshape(1, D)
    proj, s, dc = _dense_pass(tfeat_tensor, W1, V, Wv.T, wdiff)
    w, ids = _sc_call()(idx0, idx1, idx2, idx3, s, dc)
    preds = _contract_pass(w, proj, Wc, bc.reshape(1, 2))
    pseudo = _pseudo_pass(ids[:, :2].reshape(2 * G), tfeat_tensor, W1)
    return (preds, pseudo)


# speedup vs baseline: 1.0435x; 1.0161x over previous
import functools

import jax
import jax.numpy as jnp
from jax import lax
from jax.experimental import pallas as pl
from jax.experimental.pallas import tpu as pltpu
from jax.experimental.pallas import tpu_sc as plsc

N = 16384
D_IN = 1024
D = 512
G = 4
M = N // G
L = 16
BL = 512
BK = 4096


def _dense_body(x_ref, w1_ref, v_ref, wvt_ref, wct_ref,
                proj_ref, s_ref, dc_ref):
    p = jnp.maximum(
        jnp.dot(x_ref[...], w1_ref[...], preferred_element_type=jnp.float32),
        0.0)
    proj_ref[...] = p.astype(jnp.bfloat16)
    t = jnp.tanh(jnp.dot(p, v_ref[...], preferred_element_type=jnp.float32))
    s_ref[...] = jnp.sum(t * wvt_ref[...], axis=1)
    dc_ref[...] = jnp.sum(p * wct_ref[...], axis=1)


def _dense_pass(tfeat, W1, V, WvT, WcT):
    return pl.pallas_call(
        _dense_body,
        grid=(N // BL,),
        in_specs=[
            pl.BlockSpec((BL, D_IN), lambda i: (i, 0)),
            pl.BlockSpec((D_IN, D), lambda i: (0, 0)),
            pl.BlockSpec((D, D), lambda i: (0, 0)),
            pl.BlockSpec((1, D), lambda i: (0, 0)),
            pl.BlockSpec((1, D), lambda i: (0, 0)),
        ],
        out_specs=[
            pl.BlockSpec((BL, D), lambda i: (i, 0)),
            pl.BlockSpec((BL,), lambda i: (i,)),
            pl.BlockSpec((BL,), lambda i: (i,)),
        ],
        out_shape=[
            jax.ShapeDtypeStruct((N, D), jnp.bfloat16),
            jax.ShapeDtypeStruct((N,), jnp.float32),
            jax.ShapeDtypeStruct((N,), jnp.float32),
        ],
    )(tfeat, W1, V, WvT, WcT)


def _lane_gather(x, perm):
    return lax.gather(
        x, perm[:, None],
        dimension_numbers=lax.GatherDimensionNumbers(
            offset_dims=(), collapsed_slice_dims=(0,), start_index_map=(0,)),
        slice_sizes=(1,),
        mode=lax.GatherScatterMode.PROMISE_IN_BOUNDS)


def _butterfly(x, op):
    lanes = lax.iota(jnp.int32, L)
    for sh in (8, 4, 2, 1):
        x = op(x, _lane_gather(x, (lanes + sh) % L))
    return x


def _bmax(x):
    return _butterfly(x, jnp.maximum)


def _bmin(x):
    return _butterfly(x, jnp.minimum)


def _bsum(x):
    return _butterfly(x, lax.add)


WPG = 8
B = M // WPG
BCH = B // L
NSUB = 16


def _sc_body(idx0_hbm, idx1_hbm, idx2_hbm, idx3_hbm, s_hbm, dc_hbm,
             w_hbm, ids_hbm,
             idx_v, s_v, dc_v, e_v, zero_v, stg_v, stg_i, comb_v, comb_i,
             w0_sh, w1_sh, red_max, red_sum,
             red_dmax, red_rmax, red_dmin, red_rmin, sem):
    c = lax.axis_index("c")
    s = lax.axis_index("s")
    g = 2 * c + s // WPG
    slot = s % WPG
    grow = s // WPG
    base = grow * WPG * L

    def zb(j, carry):
        zero_v[pl.ds(j * L, L)] = jnp.zeros((L,), jnp.float32)
        return carry
    lax.fori_loop(0, (N // NSUB) // L, zb, 0, unroll=8)
    pltpu.sync_copy(zero_v, w0_sh.at[pl.ds(s * (N // NSUB), N // NSUB)])
    pltpu.sync_copy(zero_v, w1_sh.at[pl.ds(s * (N // NSUB), N // NSUB)])

    for gi, ref in enumerate((idx0_hbm, idx1_hbm, idx2_hbm, idx3_hbm)):
        @pl.when(g == gi)
        def _(ref=ref):
            pltpu.sync_copy(ref.at[pl.ds(slot * B, B)], idx_v)
    cp_s = pltpu.async_copy(s_hbm.at[idx_v], s_v, sem)
    cp_dc = pltpu.async_copy(dc_hbm.at[idx_v], dc_v, sem)
    cp_s.wait()
    cp_dc.wait()

    def maxbody(j, mx):
        return jnp.maximum(mx, s_v[pl.ds(j * L, L)])
    lmx = lax.fori_loop(0, BCH, maxbody,
                        jnp.full((L,), -jnp.inf, jnp.float32), unroll=8)
    stg_v[...] = lmx
    pltpu.sync_copy(stg_v, red_max.at[pl.ds(s * L, L)])
    plsc.subcore_barrier()
    pltpu.sync_copy(red_max.at[pl.ds(base, WPG * L)], comb_v)
    mxv = comb_v[pl.ds(0, L)]
    for i in range(1, WPG):
        mxv = jnp.maximum(mxv, comb_v[pl.ds(i * L, L)])
    mx = _bmax(mxv)

    def expbody(j, acc):
        e = jnp.exp(s_v[pl.ds(j * L, L)] - mx)
        e_v[pl.ds(j * L, L)] = e
        return acc + e
    lsum = lax.fori_loop(0, BCH, expbody, jnp.zeros((L,), jnp.float32),
                         unroll=8)
    stg_v[...] = lsum
    pltpu.sync_copy(stg_v, red_sum.at[pl.ds(s * L, L)])
    plsc.subcore_barrier()
    pltpu.sync_copy(red_sum.at[pl.ds(base, WPG * L)], comb_v)
    zv = comb_v[pl.ds(0, L)]
    for i in range(1, WPG):
        zv = zv + comb_v[pl.ds(i * L, L)]
    inv_z = 1.0 / _bsum(zv)

    def selbody(j, carry):
        bmaxd, bmaxr, bmind, bminr = carry
        idxc = idx_v[pl.ds(j * L, L)]
        a = e_v[pl.ds(j * L, L)] * inv_z
        e_v[pl.ds(j * L, L)] = a
        d = a * dc_v[pl.ds(j * L, L)]
        upmax = d > bmaxd
        upmin = d <= bmind
        return (jnp.where(upmax, d, bmaxd), jnp.where(upmax, idxc, bmaxr),
                jnp.where(upmin, d, bmind), jnp.where(upmin, idxc, bminr))

    init = (jnp.full((L,), -jnp.inf, jnp.float32),
            jnp.zeros((L,), jnp.int32),
            jnp.full((L,), jnp.inf, jnp.float32),
            jnp.zeros((L,), jnp.int32))
    bmaxd, bmaxr, bmind, bminr = lax.fori_loop(0, BCH, selbody, init,
                                               unroll=8)
    stg_v[...] = bmaxd
    pltpu.sync_copy(stg_v, red_dmax.at[pl.ds(s * L, L)])
    stg_i[...] = bmaxr
    pltpu.sync_copy(stg_i, red_rmax.at[pl.ds(s * L, L)])
    stg_v[...] = bmind
    pltpu.sync_copy(stg_v, red_dmin.at[pl.ds(s * L, L)])
    stg_i[...] = bminr
    pltpu.sync_copy(stg_i, red_rmin.at[pl.ds(s * L, L)])

    @pl.when(grow == 0)
    def _():
        pltpu.sync_copy(e_v, w0_sh.at[idx_v], add=True)

    @pl.when(grow == 1)
    def _():
        pltpu.sync_copy(e_v, w1_sh.at[idx_v], add=True)

    plsc.subcore_barrier()

    @pl.when(slot == 0)
    def _():
        pltpu.sync_copy(red_dmax.at[pl.ds(base, WPG * L)], comb_v)
        pltpu.sync_copy(red_rmax.at[pl.ds(base, WPG * L)], comb_i)
        bd = comb_v[pl.ds(0, L)]
        br = comb_i[pl.ds(0, L)]
        for i in range(1, WPG):
            di = comb_v[pl.ds(i * L, L)]
            ri = comb_i[pl.ds(i * L, L)]
            up = di > bd
            bd = jnp.where(up, di, bd)
            br = jnp.where(up, ri, br)
        dM = _bmax(bd)
        rmax = _bmax(jnp.where(bd == dM, br, 0))

        pltpu.sync_copy(red_dmin.at[pl.ds(base, WPG * L)], comb_v)
        pltpu.sync_copy(red_rmin.at[pl.ds(base, WPG * L)], comb_i)
        bd2 = comb_v[pl.ds(0, L)]
        br2 = comb_i[pl.ds(0, L)]
        for i in range(1, WPG):
            di = comb_v[pl.ds(i * L, L)]
            ri = comb_i[pl.ds(i * L, L)]
            up = di <= bd2
            bd2 = jnp.where(up, di, bd2)
            br2 = jnp.where(up, ri, br2)
        dm = _bmin(bd2)
        rmin = _bmax(jnp.where(bd2 == dm, br2, 0))

        lanes = lax.iota(jnp.int32, L)
        iv = jnp.where(lanes == 0, rmax, rmin)
        stg_i[...] = iv
        pltpu.sync_copy(stg_i, ids_hbm.at[g])

    CW = N // WPG

    @pl.when(grow == 0)
    def _():
        pltpu.sync_copy(w0_sh.at[pl.ds(slot * CW, CW)],
                        w_hbm.at[g, pl.ds(slot * CW, CW)])

    @pl.when(grow == 1)
    def _():
        pltpu.sync_copy(w1_sh.at[pl.ds(slot * CW, CW)],
                        w_hbm.at[g, pl.ds(slot * CW, CW)])


@functools.lru_cache(maxsize=1)
def _sc_call():
    return pl.kernel(
        _sc_body,
        out_type=[
            jax.ShapeDtypeStruct((G, N), jnp.float32),
            jax.ShapeDtypeStruct((G, L), jnp.int32),
        ],
        mesh=plsc.VectorSubcoreMesh(core_axis_name="c", subcore_axis_name="s"),
        compiler_params=pltpu.CompilerParams(needs_layout_passes=False),
        scratch_types=[
            pltpu.VMEM((B,), jnp.int32),
            pltpu.VMEM((B,), jnp.float32),
            pltpu.VMEM((B,), jnp.float32),
            pltpu.VMEM((B,), jnp.float32),
            pltpu.VMEM((N // NSUB,), jnp.float32),
            pltpu.VMEM((L,), jnp.float32),
            pltpu.VMEM((L,), jnp.int32),
            pltpu.VMEM((WPG * L,), jnp.float32),
            pltpu.VMEM((WPG * L,), jnp.int32),
            pltpu.VMEM_SHARED((N,), jnp.float32),
            pltpu.VMEM_SHARED((N,), jnp.float32),
            pltpu.VMEM_SHARED((NSUB * L,), jnp.float32),
            pltpu.VMEM_SHARED((NSUB * L,), jnp.float32),
            pltpu.VMEM_SHARED((NSUB * L,), jnp.float32),
            pltpu.VMEM_SHARED((NSUB * L,), jnp.int32),
            pltpu.VMEM_SHARED((NSUB * L,), jnp.float32),
            pltpu.VMEM_SHARED((NSUB * L,), jnp.int32),
            pltpu.SemaphoreType.DMA,
        ],
    )


def _contract_body(w_ref, proj_ref, wc_ref, bc_ref, out_ref, acc_ref):
    k = pl.program_id(0)

    @pl.when(k == 0)
    def _():
        acc_ref[...] = jnp.zeros_like(acc_ref)

    acc_ref[...] += jnp.dot(w_ref[...], proj_ref[...].astype(jnp.float32),
                            preferred_element_type=jnp.float32)

    @pl.when(k == pl.num_programs(0) - 1)
    def _():
        out_ref[...] = (jnp.dot(acc_ref[...], wc_ref[...],
                                preferred_element_type=jnp.float32)
                        + bc_ref[...])


def _contract_pass(w, proj, Wc, bc2):
    return pl.pallas_call(
        _contract_body,
        grid=(N // BK,),
        in_specs=[
            pl.BlockSpec((G, BK), lambda k: (0, k)),
            pl.BlockSpec((BK, D), lambda k: (k, 0)),
            pl.BlockSpec((D, 2), lambda k: (0, 0)),
            pl.BlockSpec((1, 2), lambda k: (0, 0)),
        ],
        out_specs=pl.BlockSpec((G, 2), lambda k: (0, 0)),
        out_shape=jax.ShapeDtypeStruct((G, 2), jnp.float32),
        scratch_shapes=[pltpu.VMEM((G, D), jnp.float32)],
    )(w, proj, Wc, bc2)


def _pseudo_body(ids_ref, tf_ref, w1_ref, o_ref, rows_v, sem):
    for i in range(2 * G):
        pltpu.make_async_copy(
            tf_ref.at[pl.ds(ids_ref[i], 1)],
            rows_v.at[pl.ds(i, 1)], sem).start()
    for i in range(2 * G):
        pltpu.make_async_copy(
            tf_ref.at[pl.ds(ids_ref[i], 1)],
            rows_v.at[pl.ds(i, 1)], sem).wait()
    res = jnp.maximum(
        jnp.dot(rows_v[...], w1_ref[...], preferred_element_type=jnp.float32),
        0.0)
    o_ref[...] = res.reshape(G, 2, D)


def _pseudo_pass(ids, tfeat, W1):
    return pl.pallas_call(
        _pseudo_body,
        in_specs=[
            pl.BlockSpec(memory_space=pltpu.SMEM),
            pl.BlockSpec(memory_space=pltpu.HBM),
            pl.BlockSpec((D_IN, D), lambda: (0, 0)),
        ],
        out_specs=pl.BlockSpec((G, 2, D), lambda: (0, 0, 0)),
        out_shape=jax.ShapeDtypeStruct((G, 2, D), jnp.float32),
        scratch_shapes=[pltpu.VMEM((2 * G, D_IN), jnp.float32),
                        pltpu.SemaphoreType.DMA],
    )(ids, tfeat, W1)


def kernel(tfeat_tensor, idx0, idx1, idx2, idx3, W1, V, Wv, Wc, bc):
    wdiff = (Wc[:, 1] - Wc[:, 0]).reshape(1, D)
    proj, s, dc = _dense_pass(tfeat_tensor, W1, V, Wv.T, wdiff)
    w, ids = _sc_call()(idx0, idx1, idx2, idx3, s, dc)
    preds = _contract_pass(w, proj, Wc, bc.reshape(1, 2))
    pseudo = _pseudo_pass(ids[:, :2].reshape(2 * G), tfeat_tensor, W1)
    return (preds, pseudo)


# speedup vs baseline: 1.0457x; 1.0022x over previous
import functools

import jax
import jax.numpy as jnp
from jax import lax
from jax.experimental import pallas as pl
from jax.experimental.pallas import tpu as pltpu
from jax.experimental.pallas import tpu_sc as plsc

N = 16384
D_IN = 1024
D = 512
G = 4
M = N // G
L = 16
BL = 512
BK = 8192


def _dense_body(x_ref, w1_ref, v_ref, wvt_ref, wct_ref,
                proj_ref, s_ref, dc_ref):
    p = jnp.maximum(
        jnp.dot(x_ref[...], w1_ref[...], preferred_element_type=jnp.float32),
        0.0)
    proj_ref[...] = p.astype(jnp.bfloat16)
    t = jnp.tanh(jnp.dot(p, v_ref[...], preferred_element_type=jnp.float32))
    s_ref[...] = jnp.sum(t * wvt_ref[...], axis=1)
    dc_ref[...] = jnp.sum(p * wct_ref[...], axis=1)


def _dense_pass(tfeat, W1, V, WvT, WcT):
    return pl.pallas_call(
        _dense_body,
        grid=(N // BL,),
        in_specs=[
            pl.BlockSpec((BL, D_IN), lambda i: (i, 0)),
            pl.BlockSpec((D_IN, D), lambda i: (0, 0)),
            pl.BlockSpec((D, D), lambda i: (0, 0)),
            pl.BlockSpec((1, D), lambda i: (0, 0)),
            pl.BlockSpec((1, D), lambda i: (0, 0)),
        ],
        out_specs=[
            pl.BlockSpec((BL, D), lambda i: (i, 0)),
            pl.BlockSpec((BL,), lambda i: (i,)),
            pl.BlockSpec((BL,), lambda i: (i,)),
        ],
        out_shape=[
            jax.ShapeDtypeStruct((N, D), jnp.bfloat16),
            jax.ShapeDtypeStruct((N,), jnp.float32),
            jax.ShapeDtypeStruct((N,), jnp.float32),
        ],
    )(tfeat, W1, V, WvT, WcT)


def _lane_gather(x, perm):
    return lax.gather(
        x, perm[:, None],
        dimension_numbers=lax.GatherDimensionNumbers(
            offset_dims=(), collapsed_slice_dims=(0,), start_index_map=(0,)),
        slice_sizes=(1,),
        mode=lax.GatherScatterMode.PROMISE_IN_BOUNDS)


def _butterfly(x, op):
    lanes = lax.iota(jnp.int32, L)
    for sh in (8, 4, 2, 1):
        x = op(x, _lane_gather(x, (lanes + sh) % L))
    return x


def _bmax(x):
    return _butterfly(x, jnp.maximum)


def _bmin(x):
    return _butterfly(x, jnp.minimum)


def _bsum(x):
    return _butterfly(x, lax.add)


WPG = 8
B = M // WPG
BCH = B // L
NSUB = 16


def _sc_body(idx0_hbm, idx1_hbm, idx2_hbm, idx3_hbm, s_hbm, dc_hbm,
             w_hbm, ids_hbm,
             idx_v, s_v, dc_v, e_v, zero_v, stg_v, stg_i, comb_v, comb_i,
             w0_sh, w1_sh, red_max, red_sum,
             red_dmax, red_rmax, red_dmin, red_rmin, sem):
    c = lax.axis_index("c")
    s = lax.axis_index("s")
    g = 2 * c + s // WPG
    slot = s % WPG
    grow = s // WPG
    base = grow * WPG * L

    def zb(j, carry):
        zero_v[pl.ds(j * L, L)] = jnp.zeros((L,), jnp.float32)
        return carry
    lax.fori_loop(0, (N // NSUB) // L, zb, 0, unroll=8)
    pltpu.sync_copy(zero_v, w0_sh.at[pl.ds(s * (N // NSUB), N // NSUB)])
    pltpu.sync_copy(zero_v, w1_sh.at[pl.ds(s * (N // NSUB), N // NSUB)])

    for gi, ref in enumerate((idx0_hbm, idx1_hbm, idx2_hbm, idx3_hbm)):
        @pl.when(g == gi)
        def _(ref=ref):
            pltpu.sync_copy(ref.at[pl.ds(slot * B, B)], idx_v)
    cp_s = pltpu.async_copy(s_hbm.at[idx_v], s_v, sem)
    cp_dc = pltpu.async_copy(dc_hbm.at[idx_v], dc_v, sem)
    cp_s.wait()
    cp_dc.wait()

    def maxbody(j, mx):
        return jnp.maximum(mx, s_v[pl.ds(j * L, L)])
    lmx = lax.fori_loop(0, BCH, maxbody,
                        jnp.full((L,), -jnp.inf, jnp.float32), unroll=8)
    stg_v[...] = lmx
    pltpu.sync_copy(stg_v, red_max.at[pl.ds(s * L, L)])
    plsc.subcore_barrier()
    pltpu.sync_copy(red_max.at[pl.ds(base, WPG * L)], comb_v)
    mxv = comb_v[pl.ds(0, L)]
    for i in range(1, WPG):
        mxv = jnp.maximum(mxv, comb_v[pl.ds(i * L, L)])
    mx = _bmax(mxv)

    def expbody(j, acc):
        e = jnp.exp(s_v[pl.ds(j * L, L)] - mx)
        e_v[pl.ds(j * L, L)] = e
        return acc + e
    lsum = lax.fori_loop(0, BCH, expbody, jnp.zeros((L,), jnp.float32),
                         unroll=8)
    stg_v[...] = lsum
    pltpu.sync_copy(stg_v, red_sum.at[pl.ds(s * L, L)])
    plsc.subcore_barrier()
    pltpu.sync_copy(red_sum.at[pl.ds(base, WPG * L)], comb_v)
    zv = comb_v[pl.ds(0, L)]
    for i in range(1, WPG):
        zv = zv + comb_v[pl.ds(i * L, L)]
    inv_z = 1.0 / _bsum(zv)

    def selbody(j, carry):
        bmaxd, bmaxr, bmind, bminr = carry
        idxc = idx_v[pl.ds(j * L, L)]
        a = e_v[pl.ds(j * L, L)] * inv_z
        e_v[pl.ds(j * L, L)] = a
        d = a * dc_v[pl.ds(j * L, L)]
        upmax = d > bmaxd
        upmin = d <= bmind
        return (jnp.where(upmax, d, bmaxd), jnp.where(upmax, idxc, bmaxr),
                jnp.where(upmin, d, bmind), jnp.where(upmin, idxc, bminr))

    init = (jnp.full((L,), -jnp.inf, jnp.float32),
            jnp.zeros((L,), jnp.int32),
            jnp.full((L,), jnp.inf, jnp.float32),
            jnp.zeros((L,), jnp.int32))
    bmaxd, bmaxr, bmind, bminr = lax.fori_loop(0, BCH, selbody, init,
                                               unroll=8)
    stg_v[...] = bmaxd
    pltpu.sync_copy(stg_v, red_dmax.at[pl.ds(s * L, L)])
    stg_i[...] = bmaxr
    pltpu.sync_copy(stg_i, red_rmax.at[pl.ds(s * L, L)])
    stg_v[...] = bmind
    pltpu.sync_copy(stg_v, red_dmin.at[pl.ds(s * L, L)])
    stg_i[...] = bminr
    pltpu.sync_copy(stg_i, red_rmin.at[pl.ds(s * L, L)])

    @pl.when(grow == 0)
    def _():
        pltpu.sync_copy(e_v, w0_sh.at[idx_v], add=True)

    @pl.when(grow == 1)
    def _():
        pltpu.sync_copy(e_v, w1_sh.at[idx_v], add=True)

    plsc.subcore_barrier()

    @pl.when(slot == 0)
    def _():
        pltpu.sync_copy(red_dmax.at[pl.ds(base, WPG * L)], comb_v)
        pltpu.sync_copy(red_rmax.at[pl.ds(base, WPG * L)], comb_i)
        bd = comb_v[pl.ds(0, L)]
        br = comb_i[pl.ds(0, L)]
        for i in range(1, WPG):
            di = comb_v[pl.ds(i * L, L)]
            ri = comb_i[pl.ds(i * L, L)]
            up = di > bd
            bd = jnp.where(up, di, bd)
            br = jnp.where(up, ri, br)
        dM = _bmax(bd)
        rmax = _bmax(jnp.where(bd == dM, br, 0))

        pltpu.sync_copy(red_dmin.at[pl.ds(base, WPG * L)], comb_v)
        pltpu.sync_copy(red_rmin.at[pl.ds(base, WPG * L)], comb_i)
        bd2 = comb_v[pl.ds(0, L)]
        br2 = comb_i[pl.ds(0, L)]
        for i in range(1, WPG):
            di = comb_v[pl.ds(i * L, L)]
            ri = comb_i[pl.ds(i * L, L)]
            up = di <= bd2
            bd2 = jnp.where(up, di, bd2)
            br2 = jnp.where(up, ri, br2)
        dm = _bmin(bd2)
        rmin = _bmax(jnp.where(bd2 == dm, br2, 0))

        lanes = lax.iota(jnp.int32, L)
        iv = jnp.where(lanes == 0, rmax, rmin)
        stg_i[...] = iv
        pltpu.sync_copy(stg_i, ids_hbm.at[g])

    CW = N // WPG

    @pl.when(grow == 0)
    def _():
        pltpu.sync_copy(w0_sh.at[pl.ds(slot * CW, CW)],
                        w_hbm.at[g, pl.ds(slot * CW, CW)])

    @pl.when(grow == 1)
    def _():
        pltpu.sync_copy(w1_sh.at[pl.ds(slot * CW, CW)],
                        w_hbm.at[g, pl.ds(slot * CW, CW)])


@functools.lru_cache(maxsize=1)
def _sc_call():
    return pl.kernel(
        _sc_body,
        out_type=[
            jax.ShapeDtypeStruct((G, N), jnp.float32),
            jax.ShapeDtypeStruct((G, L), jnp.int32),
        ],
        mesh=plsc.VectorSubcoreMesh(core_axis_name="c", subcore_axis_name="s"),
        compiler_params=pltpu.CompilerParams(needs_layout_passes=False),
        scratch_types=[
            pltpu.VMEM((B,), jnp.int32),
            pltpu.VMEM((B,), jnp.float32),
            pltpu.VMEM((B,), jnp.float32),
            pltpu.VMEM((B,), jnp.float32),
            pltpu.VMEM((N // NSUB,), jnp.float32),
            pltpu.VMEM((L,), jnp.float32),
            pltpu.VMEM((L,), jnp.int32),
            pltpu.VMEM((WPG * L,), jnp.float32),
            pltpu.VMEM((WPG * L,), jnp.int32),
            pltpu.VMEM_SHARED((N,), jnp.float32),
            pltpu.VMEM_SHARED((N,), jnp.float32),
            pltpu.VMEM_SHARED((NSUB * L,), jnp.float32),
            pltpu.VMEM_SHARED((NSUB * L,), jnp.float32),
            pltpu.VMEM_SHARED((NSUB * L,), jnp.float32),
            pltpu.VMEM_SHARED((NSUB * L,), jnp.int32),
            pltpu.VMEM_SHARED((NSUB * L,), jnp.float32),
            pltpu.VMEM_SHARED((NSUB * L,), jnp.int32),
            pltpu.SemaphoreType.DMA,
        ],
    )


def _contract_body(w_ref, proj_ref, wc_ref, bc_ref, out_ref, acc_ref):
    k = pl.program_id(0)

    @pl.when(k == 0)
    def _():
        acc_ref[...] = jnp.zeros_like(acc_ref)

    acc_ref[...] += jnp.dot(w_ref[...], proj_ref[...].astype(jnp.float32),
                            preferred_element_type=jnp.float32)

    @pl.when(k == pl.num_programs(0) - 1)
    def _():
        out_ref[...] = (jnp.dot(acc_ref[...], wc_ref[...],
                                preferred_element_type=jnp.float32)
                        + bc_ref[...])


def _contract_pass(w, proj, Wc, bc2):
    return pl.pallas_call(
        _contract_body,
        grid=(N // BK,),
        in_specs=[
            pl.BlockSpec((G, BK), lambda k: (0, k)),
            pl.BlockSpec((BK, D), lambda k: (k, 0)),
            pl.BlockSpec((D, 2), lambda k: (0, 0)),
            pl.BlockSpec((1, 2), lambda k: (0, 0)),
        ],
        out_specs=pl.BlockSpec((G, 2), lambda k: (0, 0)),
        out_shape=jax.ShapeDtypeStruct((G, 2), jnp.float32),
        scratch_shapes=[pltpu.VMEM((G, D), jnp.float32)],
    )(w, proj, Wc, bc2)


def _pseudo_body(ids_ref, tf_ref, w1_ref, o_ref, rows_v, sem):
    for i in range(2 * G):
        pltpu.make_async_copy(
            tf_ref.at[pl.ds(ids_ref[i], 1)],
            rows_v.at[pl.ds(i, 1)], sem).start()
    for i in range(2 * G):
        pltpu.make_async_copy(
            tf_ref.at[pl.ds(ids_ref[i], 1)],
            rows_v.at[pl.ds(i, 1)], sem).wait()
    res = jnp.maximum(
        jnp.dot(rows_v[...], w1_ref[...], preferred_element_type=jnp.float32),
        0.0)
    o_ref[...] = res.reshape(G, 2, D)


def _pseudo_pass(ids, tfeat, W1):
    return pl.pallas_call(
        _pseudo_body,
        in_specs=[
            pl.BlockSpec(memory_space=pltpu.SMEM),
            pl.BlockSpec(memory_space=pltpu.HBM),
            pl.BlockSpec((D_IN, D), lambda: (0, 0)),
        ],
        out_specs=pl.BlockSpec((G, 2, D), lambda: (0, 0, 0)),
        out_shape=jax.ShapeDtypeStruct((G, 2, D), jnp.float32),
        scratch_shapes=[pltpu.VMEM((2 * G, D_IN), jnp.float32),
                        pltpu.SemaphoreType.DMA],
    )(ids, tfeat, W1)


def kernel(tfeat_tensor, idx0, idx1, idx2, idx3, W1, V, Wv, Wc, bc):
    wdiff = (Wc[:, 1] - Wc[:, 0]).reshape(1, D)
    proj, s, dc = _dense_pass(tfeat_tensor, W1, V, Wv.T, wdiff)
    w, ids = _sc_call()(idx0, idx1, idx2, idx3, s, dc)
    preds = _contract_pass(w, proj, Wc, bc.reshape(1, 2))
    pseudo = _pseudo_pass(ids[:, :2].reshape(2 * G), tfeat_tensor, W1)
    return (preds, pseudo)
